# baseline (speedup 1.0000x reference)
import functools

import jax
import jax.numpy as jnp
from jax import lax
from jax.experimental import pallas as pl
from jax.experimental.pallas import tpu as pltpu
from jax.experimental.pallas import tpu_sc as plsc

EMB = 64
SCALE = 8.0
NC = 2
NS = 16
L = 16
NW = NC * NS
CHUNK = 128
ET = EMB // 8


@functools.lru_cache(maxsize=None)
def _make(n_s):
    mesh = plsc.VectorSubcoreMesh(
        core_axis_name="c", subcore_axis_name="s",
        num_cores=NC, num_subcores=NS)

    def body(tok_hbm, table_hbm, out_hbm, idx_v, rows0, rows1, buf, sem0, sem1):
        wid = lax.axis_index("s") * NC + lax.axis_index("c")
        pltpu.sync_copy(tok_hbm.at[:, wid], idx_v)
        lane = lax.broadcasted_iota(jnp.int32, (L,), 0)

        def fire(s, rows, sem):
            pltpu.async_copy(table_hbm.at[idx_v.at[s]], rows, sem)

        def drain(s, rows, sem):
            pltpu.make_async_copy(table_hbm.at[idx_v.at[s]], rows, sem).wait()

        def emit(s, rows, sem):
            drain(s, rows, sem)

            @pl.loop(0, EMB)
            def _(e):
                col = jnp.full((L,), e, jnp.int32)
                for j in range(CHUNK // L):
                    v = plsc.load_gather(rows, [lane + (j * L), col])
                    buf[e // 8, e % 8, pl.ds(j * L, L)] = v * SCALE

            pltpu.sync_copy(buf, out_hbm.at[s, :, wid])

        fire(0, rows0, sem0)

        @pl.loop(0, n_s - 2, step=2)
        def _(so):
            fire(so + 1, rows1, sem1)
            emit(so, rows0, sem0)
            fire(so + 2, rows0, sem0)
            emit(so + 1, rows1, sem1)

        fire(n_s - 1, rows1, sem1)
        emit(n_s - 2, rows0, sem0)
        emit(n_s - 1, rows1, sem1)

    return pl.kernel(
        body,
        out_type=jax.ShapeDtypeStruct((n_s, ET, NW, 8, CHUNK), jnp.float32),
        mesh=mesh,
        compiler_params=pltpu.CompilerParams(
            use_tc_tiling_on_sc=False, needs_layout_passes=False),
        scratch_types=[
            pltpu.VMEM((n_s, CHUNK), jnp.int32),
            pltpu.VMEM((CHUNK, EMB), jnp.float32),
            pltpu.VMEM((CHUNK, EMB), jnp.float32),
            pltpu.VMEM((ET, 8, CHUNK), jnp.float32),
            pltpu.SemaphoreType.DMA,
            pltpu.SemaphoreType.DMA,
        ],
    )


def kernel(tokens, table):
    bt, n_s = tokens.shape
    tok = tokens.astype(jnp.int32).T.reshape(n_s, NW, CHUNK)
    out5 = _make(n_s)(tok, table)
    return out5.transpose(2, 4, 0, 1, 3).reshape(bt, n_s, EMB)
